# Initial kernel scaffold; baseline (speedup 1.0000x reference)
import functools

import jax
import jax.numpy as jnp
from jax import lax
from jax.experimental import pallas as pl
from jax.experimental.pallas import tpu as pltpu
from jax.experimental.pallas import tpu_sc as plsc

B = 16384 * 20
D = 32
NC, NS = 2, 16
NW = NC * NS
B_PER_W = B // NW
CH = 1024
NCHUNK = B_PER_W // CH

_mesh = plsc.VectorSubcoreMesh(core_axis_name="c", subcore_axis_name="s")


@functools.partial(
    pl.kernel,
    mesh=_mesh,
    out_type=jax.ShapeDtypeStruct((B, D), jnp.float32),
    scratch_types=[
        pltpu.VMEM((NCHUNK, CH), jnp.int32),
        pltpu.VMEM((2, CH, D), jnp.float32),
        pltpu.SemaphoreType.DMA,
        pltpu.SemaphoreType.DMA,
        pltpu.SemaphoreType.DMA,
        pltpu.SemaphoreType.DMA,
    ],
)
def _gather_kernel(idx_hbm, table_hbm, out_hbm, idx_v, rows_v,
                   sem_g0, sem_g1, sem_w0, sem_w1):
    wid = lax.axis_index("s") * NC + lax.axis_index("c")
    base = wid * B_PER_W
    sem_g = (sem_g0, sem_g1)
    sem_w = (sem_w0, sem_w1)

    pltpu.sync_copy(idx_hbm.at[wid], idx_v)

    h_g = [None, None]
    h_w = [None, None]
    h_g[0] = pltpu.async_copy(table_hbm.at[idx_v.at[0]], rows_v.at[0], sem_g[0])
    for c in range(NCHUNK):
        b = c % 2
        nb = (c + 1) % 2
        if c + 1 < NCHUNK:
            if h_w[nb] is not None:
                h_w[nb].wait()
                h_w[nb] = None
            h_g[nb] = pltpu.async_copy(
                table_hbm.at[idx_v.at[c + 1]], rows_v.at[nb], sem_g[nb])
        h_g[b].wait()
        h_w[b] = pltpu.async_copy(
            rows_v.at[b], out_hbm.at[pl.ds(base + c * CH, CH)], sem_w[b])
    for b in range(2):
        if h_w[b] is not None:
            h_w[b].wait()


def kernel(index, table):
    idx = index.reshape(-1).astype(jnp.int32).reshape(NW, NCHUNK, CH)
    out = _gather_kernel(idx, table)
    return out.reshape(index.shape[0], index.shape[1], D)


# speedup vs baseline: 1.5148x; 1.5148x over previous
import functools

import jax
import jax.numpy as jnp
from jax import lax
from jax.experimental import pallas as pl
from jax.experimental.pallas import tpu as pltpu
from jax.experimental.pallas import tpu_sc as plsc

B = 16384 * 20
D = 32
NC, NS = 2, 16
NW = NC * NS
B_PER_W = B // NW
CH = 1024
NCHUNK = B_PER_W // CH

_mesh = plsc.VectorSubcoreMesh(core_axis_name="c", subcore_axis_name="s")


@functools.partial(
    pl.kernel,
    mesh=_mesh,
    out_type=jax.ShapeDtypeStruct((B, D), jnp.float32),
    scratch_types=[
        pltpu.VMEM((NCHUNK, CH), jnp.int32),
        pltpu.VMEM((2, CH, D), jnp.float32),
        pltpu.SemaphoreType.DMA,
        pltpu.SemaphoreType.DMA,
        pltpu.SemaphoreType.DMA,
        pltpu.SemaphoreType.DMA,
    ],
    compiler_params=pltpu.CompilerParams(use_tc_tiling_on_sc=False),
)
def _gather_kernel(idx_hbm, table_hbm, out_hbm, idx_v, rows_v,
                   sem_g0, sem_g1, sem_w0, sem_w1):
    wid = lax.axis_index("s") * NC + lax.axis_index("c")
    base = wid * B_PER_W
    sem_g = (sem_g0, sem_g1)
    sem_w = (sem_w0, sem_w1)

    pltpu.sync_copy(idx_hbm.at[wid], idx_v)

    h_g = [None, None]
    h_w = [None, None]
    h_g[0] = pltpu.async_copy(table_hbm.at[idx_v.at[0]], rows_v.at[0], sem_g[0])
    for c in range(NCHUNK):
        b = c % 2
        nb = (c + 1) % 2
        if c + 1 < NCHUNK:
            if h_w[nb] is not None:
                h_w[nb].wait()
                h_w[nb] = None
            h_g[nb] = pltpu.async_copy(
                table_hbm.at[idx_v.at[c + 1]], rows_v.at[nb], sem_g[nb])
        h_g[b].wait()
        h_w[b] = pltpu.async_copy(
            rows_v.at[b], out_hbm.at[pl.ds(base + c * CH, CH)], sem_w[b])
    for b in range(2):
        if h_w[b] is not None:
            h_w[b].wait()


def kernel(index, table):
    idx = index.reshape(-1).astype(jnp.int32).reshape(NW, NCHUNK, CH)
    out = _gather_kernel(idx, table)
    return out.reshape(index.shape[0], index.shape[1], D)
